# baseline (speedup 1.0000x reference)
import functools

import jax
import jax.numpy as jnp
from jax import lax
from jax.experimental import pallas as pl
from jax.experimental.pallas import tpu as pltpu
from jax.experimental.pallas import tpu_sc as plsc
from jax.experimental.layout import Format, Layout, with_layout_constraint

B = 4096
S = 200
H = 64
NUM_INST = 100000
NUM_INST_PAD = 100096
NUM_SESS = 1000
CW = 256
SW = 100


def _make_inst_kernel():
    info = plsc.get_sparse_core_info()
    nc, ns = info.num_cores, info.num_subcores
    nw = nc * ns
    rows_per_w = H // nw

    mesh = plsc.VectorSubcoreMesh(core_axis_name="c", subcore_axis_name="s")

    @functools.partial(
        pl.kernel,
        mesh=mesh,
        out_type=jax.ShapeDtypeStruct((H, B), jnp.float32),
        compiler_params=pltpu.CompilerParams(
            use_tc_tiling_on_sc=False, needs_layout_passes=False),
        scratch_types=[
            pltpu.VMEM((NUM_INST_PAD,), jnp.float32),
            pltpu.VMEM((B,), jnp.int32),
            pltpu.VMEM((B,), jnp.float32),
        ],
    )
    def k(itab_hbm, iids_hbm, iemb_hbm, row_v, iid_v, orow_v):
        wid = lax.axis_index("s") * nc + lax.axis_index("c")
        pltpu.sync_copy(iids_hbm, iid_v)
        for t in range(rows_per_w):
            h = wid * rows_per_w + t
            pltpu.sync_copy(itab_hbm.at[h], row_v)

            @plsc.parallel_loop(0, B // 16, step=1, unroll=4)
            def c_body(c):
                idxv = iid_v[pl.ds(16 * c, 16)]
                orow_v[pl.ds(16 * c, 16)] = plsc.load_gather(row_v, [idxv])

            pltpu.sync_copy(orow_v, iemb_hbm.at[h])

    return k


def _make_main_kernel():
    info = plsc.get_sparse_core_info()
    nc, ns = info.num_cores, info.num_subcores
    nw = nc * ns
    cols_per_w = B // nw

    mesh = plsc.VectorSubcoreMesh(core_axis_name="c", subcore_axis_name="s")

    @functools.partial(
        pl.kernel,
        mesh=mesh,
        out_type=jax.ShapeDtypeStruct((S, H, B), jnp.float32),
        compiler_params=pltpu.CompilerParams(
            use_tc_tiling_on_sc=False, needs_layout_passes=False),
        scratch_types=[
            pltpu.VMEM((H, NUM_SESS), jnp.float32),
            pltpu.VMEM((H, CW), jnp.float32),
            pltpu.VMEM((SW // 2, CW), jnp.int32),
            pltpu.VMEM((H, CW), jnp.float32),
            pltpu.VMEM((H, CW), jnp.float32),
            pltpu.SemaphoreType.DMA,
            pltpu.SemaphoreType.DMA,
            pltpu.SemaphoreType.DMA,
            pltpu.SemaphoreType.DMA,
        ],
    )
    def k(hid_hbm, sid_hbm, stab_hbm, iemb_hbm, out_hbm,
          table_v, iemb_v, sid_v, hid0, hid1, hsem0, hsem1, osem0, osem1):
        wid = lax.axis_index("s") * nc + lax.axis_index("c")
        c0 = lax.rem(wid, B // CW) * CW
        s0 = (wid // (B // CW)) * SW

        bufs = ((hid0, hsem0, osem0), (hid1, hsem1, osem1))

        pltpu.sync_copy(stab_hbm, table_v)
        pltpu.sync_copy(iemb_hbm.at[:, pl.ds(c0, CW)], iemb_v)
        pltpu.sync_copy(sid_hbm.at[pl.ds(s0, SW // 2), pl.ds(c0, CW)], sid_v)

        def issue_in(i, hid, hsem):
            pltpu.async_copy(hid_hbm.at[s0 + i, :, pl.ds(c0, CW)], hid, hsem)

        issue_in(0, hid0, hsem0)

        def compute(i, hid):
            ls = lax.rem(i, SW // 2)
            nch = CW // 16

            @plsc.parallel_loop(0, H * nch, step=1, unroll=8)
            def hc_body(it):
                h = it // nch
                co = 16 * lax.rem(it, nch)
                hv = jnp.full((16,), h, jnp.int32)
                idxv = sid_v[ls, pl.ds(co, 16)]
                srow = plsc.load_gather(table_v, [hv, idxv])
                plsc.addupdate(
                    hid.at[h, pl.ds(co, 16)],
                    srow + iemb_v[h, pl.ds(co, 16)])

        def pair_body(g, _):
            @pl.when(g == SW // 4)
            def _():
                pltpu.sync_copy(
                    sid_hbm.at[pl.ds(s0 + SW // 2, SW // 2), pl.ds(c0, CW)],
                    sid_v)

            for j in (0, 1):
                hid, hsem, osem = bufs[j]
                nhid, nhsem, nosem = bufs[1 - j]
                i = 2 * g + j

                @pl.when(i >= 1)
                def _():
                    pltpu.make_async_copy(
                        nhid, out_hbm.at[0, :, pl.ds(0, CW)], nosem).wait()

                @pl.when(i + 1 < SW)
                def _():
                    issue_in(i + 1, nhid, nhsem)

                pltpu.make_async_copy(
                    hid_hbm.at[0, :, pl.ds(0, CW)], hid, hsem).wait()
                compute(i, hid)
                pltpu.async_copy(
                    hid, out_hbm.at[s0 + i, :, pl.ds(c0, CW)], osem)
            return 0

        lax.fori_loop(0, SW // 2, pair_body, 0)
        pltpu.make_async_copy(
            hid1, out_hbm.at[0, :, pl.ds(0, CW)], osem1).wait()

    return k


_inst_call = None
_main_call = None


def kernel(hidden_states, instrument_ids, session_ids, instrument_table, session_table):
    global _inst_call, _main_call
    if _inst_call is None:
        _inst_call = _make_inst_kernel()
        _main_call = _make_main_kernel()
    hid_t = jnp.transpose(hidden_states, (1, 2, 0))
    sid_t = jnp.transpose(session_ids.astype(jnp.int32), (1, 0))
    itab_t = jnp.pad(jnp.transpose(instrument_table, (1, 0)),
                     ((0, 0), (0, NUM_INST_PAD - NUM_INST)))
    stab_t = jnp.transpose(session_table, (1, 0))
    iids = instrument_ids.astype(jnp.int32)
    iemb = _inst_call(itab_t, iids)
    out_t = _main_call(hid_t, sid_t, stab_t, iemb)
    out = jnp.transpose(out_t, (2, 0, 1))
    return with_layout_constraint(
        out, Layout(major_to_minor=(1, 2, 0), tiling=((8, 128),)))


# speedup vs baseline: 1.0891x; 1.0891x over previous
import functools

import jax
import jax.numpy as jnp
from jax import lax
from jax.experimental import pallas as pl
from jax.experimental.pallas import tpu as pltpu
from jax.experimental.pallas import tpu_sc as plsc

B = 4096
S = 200
H = 64
NUM_INST = 100000
NUM_SESS = 1000


def _make_inst_kernel():
    info = plsc.get_sparse_core_info()
    nc, ns = info.num_cores, info.num_subcores
    nw = nc * ns
    rows_per_w = H // nw

    mesh = plsc.VectorSubcoreMesh(core_axis_name="c", subcore_axis_name="s")

    @functools.partial(
        pl.kernel,
        mesh=mesh,
        out_type=jax.ShapeDtypeStruct((H, B), jnp.float32),
        compiler_params=pltpu.CompilerParams(
            use_tc_tiling_on_sc=False, needs_layout_passes=False),
        scratch_types=[
            pltpu.VMEM((NUM_INST,), jnp.float32),
            pltpu.VMEM((B,), jnp.int32),
            pltpu.VMEM((B,), jnp.float32),
        ],
    )
    def k(itab_hbm, iids_hbm, iemb_hbm, row_v, iid_v, orow_v):
        wid = lax.axis_index("s") * nc + lax.axis_index("c")
        pltpu.sync_copy(iids_hbm, iid_v)
        for t in range(rows_per_w):
            h = wid * rows_per_w + t
            pltpu.sync_copy(itab_hbm.at[h], row_v)

            @plsc.parallel_loop(0, B // 16, step=1, unroll=4)
            def c_body(c):
                idxv = iid_v[pl.ds(16 * c, 16)]
                orow_v[pl.ds(16 * c, 16)] = plsc.load_gather(row_v, [idxv])

            pltpu.sync_copy(orow_v, iemb_hbm.at[h])

    return k


def _make_main_kernel():
    info = plsc.get_sparse_core_info()
    nc, ns = info.num_cores, info.num_subcores
    nw = nc * ns
    cols_per_w = B // nw

    mesh = plsc.VectorSubcoreMesh(core_axis_name="c", subcore_axis_name="s")

    @functools.partial(
        pl.kernel,
        mesh=mesh,
        out_type=jax.ShapeDtypeStruct((S, H, B), jnp.float32),
        compiler_params=pltpu.CompilerParams(
            use_tc_tiling_on_sc=False, needs_layout_passes=False),
        scratch_types=[
            pltpu.VMEM((H, NUM_SESS), jnp.float32),
            pltpu.VMEM((H, 128), jnp.float32),
            pltpu.VMEM((S, 128), jnp.int32),
            pltpu.VMEM((H, 128), jnp.float32),
            pltpu.VMEM((H, 128), jnp.float32),
            pltpu.SemaphoreType.DMA,
            pltpu.SemaphoreType.DMA,
            pltpu.SemaphoreType.DMA,
            pltpu.SemaphoreType.DMA,
        ],
    )
    def k(hid_hbm, sid_hbm, stab_hbm, iemb_hbm, out_hbm,
          table_v, iemb_v, sid_v, hid0, hid1, hsem0, hsem1, osem0, osem1):
        wid = lax.axis_index("s") * nc + lax.axis_index("c")
        c0 = wid * cols_per_w

        bufs = ((hid0, hsem0, osem0), (hid1, hsem1, osem1))

        pltpu.sync_copy(stab_hbm, table_v)
        pltpu.sync_copy(iemb_hbm.at[:, pl.ds(c0, 128)], iemb_v)
        pltpu.sync_copy(sid_hbm.at[:, pl.ds(c0, 128)], sid_v)

        def issue_in(s, hid, hsem):
            pltpu.async_copy(hid_hbm.at[s, :, pl.ds(c0, 128)], hid, hsem)

        issue_in(0, hid0, hsem0)

        def compute(s, hid):
            idxs = [sid_v[s, pl.ds(16 * c, 16)] for c in range(8)]

            @plsc.parallel_loop(0, H, step=1, unroll=2)
            def h_body(h):
                hv = jnp.full((16,), h, jnp.int32)
                for c in range(8):
                    srow = plsc.load_gather(table_v, [hv, idxs[c]])
                    plsc.addupdate(
                        hid.at[h, pl.ds(16 * c, 16)],
                        srow + iemb_v[h, pl.ds(16 * c, 16)])

        def pair_body(g, _):
            for j in (0, 1):
                hid, hsem, osem = bufs[j]
                nhid, nhsem, nosem = bufs[1 - j]
                s = 2 * g + j

                @pl.when(s >= 1)
                def _():
                    pltpu.make_async_copy(
                        nhid, out_hbm.at[0, :, pl.ds(0, 128)], nosem).wait()

                @pl.when(s + 1 < S)
                def _():
                    issue_in(s + 1, nhid, nhsem)

                pltpu.make_async_copy(
                    hid_hbm.at[0, :, pl.ds(0, 128)], hid, hsem).wait()
                compute(s, hid)
                pltpu.async_copy(
                    hid, out_hbm.at[s, :, pl.ds(c0, 128)], osem)
            return 0

        lax.fori_loop(0, S // 2, pair_body, 0)
        pltpu.make_async_copy(
            hid1, out_hbm.at[0, :, pl.ds(0, 128)], osem1).wait()

    return k


_inst_call = None
_main_call = None


def kernel(hidden_states, instrument_ids, session_ids, instrument_table, session_table):
    global _inst_call, _main_call
    if _inst_call is None:
        _inst_call = _make_inst_kernel()
        _main_call = _make_main_kernel()
    hid_t = jnp.transpose(hidden_states, (1, 2, 0))
    sid_t = jnp.transpose(session_ids.astype(jnp.int32), (1, 0))
    itab_t = jnp.transpose(instrument_table, (1, 0))
    stab_t = jnp.transpose(session_table, (1, 0))
    iids = instrument_ids.astype(jnp.int32)
    iemb = _inst_call(itab_t, iids)
    out_t = _main_call(hid_t, sid_t, stab_t, iemb)
    return jnp.transpose(out_t, (2, 0, 1))


# speedup vs baseline: 1.2103x; 1.1113x over previous
import functools

import jax
import jax.numpy as jnp
from jax import lax
from jax.experimental import pallas as pl
from jax.experimental.pallas import tpu as pltpu
from jax.experimental.pallas import tpu_sc as plsc

B = 4096
S = 200
H = 64
NUM_INST = 100000
NUM_SESS = 1000


def _make_inst_kernel():
    info = plsc.get_sparse_core_info()
    nc, ns = info.num_cores, info.num_subcores
    nw = nc * ns
    rows_per_w = H // nw

    mesh = plsc.VectorSubcoreMesh(core_axis_name="c", subcore_axis_name="s")

    @functools.partial(
        pl.kernel,
        mesh=mesh,
        out_type=jax.ShapeDtypeStruct((H, B), jnp.float32),
        compiler_params=pltpu.CompilerParams(
            use_tc_tiling_on_sc=False, needs_layout_passes=False),
        scratch_types=[
            pltpu.VMEM((NUM_INST,), jnp.float32),
            pltpu.VMEM((B,), jnp.int32),
            pltpu.VMEM((B,), jnp.float32),
        ],
    )
    def k(itab_hbm, iids_hbm, iemb_hbm, row_v, iid_v, orow_v):
        wid = lax.axis_index("s") * nc + lax.axis_index("c")
        pltpu.sync_copy(iids_hbm, iid_v)
        for t in range(rows_per_w):
            h = wid * rows_per_w + t
            pltpu.sync_copy(itab_hbm.at[h], row_v)

            @plsc.parallel_loop(0, B // 16, step=1, unroll=4)
            def c_body(c):
                idxv = iid_v[pl.ds(16 * c, 16)]
                orow_v[pl.ds(16 * c, 16)] = plsc.load_gather(row_v, [idxv])

            pltpu.sync_copy(orow_v, iemb_hbm.at[h])

    return k


def _make_main_kernel():
    info = plsc.get_sparse_core_info()
    nc, ns = info.num_cores, info.num_subcores
    nw = nc * ns
    cols_per_w = B // nw

    mesh = plsc.VectorSubcoreMesh(core_axis_name="c", subcore_axis_name="s")

    @functools.partial(
        pl.kernel,
        mesh=mesh,
        out_type=jax.ShapeDtypeStruct((S, H, B), jnp.float32),
        compiler_params=pltpu.CompilerParams(
            use_tc_tiling_on_sc=False, needs_layout_passes=False),
        scratch_types=[
            pltpu.VMEM((H, NUM_SESS), jnp.float32),
            pltpu.VMEM((H, 128), jnp.float32),
            pltpu.VMEM((S // 2, 128), jnp.int32),
            pltpu.VMEM((H, 128), jnp.float32),
            pltpu.VMEM((H, 128), jnp.float32),
            pltpu.VMEM((H, 128), jnp.float32),
            pltpu.VMEM((H, 128), jnp.float32),
            pltpu.SemaphoreType.DMA,
            pltpu.SemaphoreType.DMA,
            pltpu.SemaphoreType.DMA,
            pltpu.SemaphoreType.DMA,
            pltpu.SemaphoreType.DMA,
            pltpu.SemaphoreType.DMA,
            pltpu.SemaphoreType.DMA,
            pltpu.SemaphoreType.DMA,
        ],
    )
    def k(hid_hbm, sid_hbm, stab_hbm, iemb_hbm, out_hbm,
          table_v, iemb_v, sid_v, hid0, hid1, hid2, hid3,
          hsem0, hsem1, hsem2, hsem3, osem0, osem1, osem2, osem3):
        wid = lax.axis_index("s") * nc + lax.axis_index("c")
        c0 = wid * cols_per_w

        bufs = ((hid0, hsem0, osem0), (hid1, hsem1, osem1),
                (hid2, hsem2, osem2), (hid3, hsem3, osem3))

        pltpu.sync_copy(stab_hbm, table_v)
        pltpu.sync_copy(iemb_hbm.at[:, pl.ds(c0, 128)], iemb_v)
        pltpu.sync_copy(sid_hbm.at[pl.ds(0, S // 2), pl.ds(c0, 128)], sid_v)

        def issue_in(s, hid, hsem):
            pltpu.async_copy(hid_hbm.at[s, :, pl.ds(c0, 128)], hid, hsem)

        def wait_out(hid, osem):
            pltpu.make_async_copy(
                hid, out_hbm.at[0, :, pl.ds(0, 128)], osem).wait()

        issue_in(0, hid0, hsem0)
        issue_in(1, hid1, hsem1)

        def compute(s, hid):
            ls = lax.rem(s, S // 2)
            idxs = [sid_v[ls, pl.ds(16 * c, 16)] for c in range(8)]

            @plsc.parallel_loop(0, H, step=1, unroll=2)
            def h_body(h):
                hv = jnp.full((16,), h, jnp.int32)
                for c in range(8):
                    srow = plsc.load_gather(table_v, [hv, idxs[c]])
                    plsc.addupdate(
                        hid.at[h, pl.ds(16 * c, 16)],
                        srow + iemb_v[h, pl.ds(16 * c, 16)])

        def group_body(g, _):
            @pl.when(g == S // 8)
            def _():
                pltpu.sync_copy(
                    sid_hbm.at[pl.ds(S // 2, S // 2), pl.ds(c0, 128)], sid_v)

            for j in range(4):
                hid, hsem, osem = bufs[j]
                phid, phsem, posem = bufs[(j + 2) % 4]
                s = 4 * g + j

                @pl.when(s + 2 < S)
                def _():
                    @pl.when(s >= 2)
                    def _():
                        wait_out(phid, posem)
                    issue_in(s + 2, phid, phsem)

                pltpu.make_async_copy(
                    hid_hbm.at[0, :, pl.ds(0, 128)], hid, hsem).wait()
                compute(s, hid)
                pltpu.async_copy(
                    hid, out_hbm.at[s, :, pl.ds(c0, 128)], osem)
            return 0

        lax.fori_loop(0, S // 4, group_body, 0)
        for j in range(4):
            hid, _, osem = bufs[j]
            wait_out(hid, osem)

    return k


_inst_call = None
_main_call = None


def kernel(hidden_states, instrument_ids, session_ids, instrument_table, session_table):
    global _inst_call, _main_call
    if _inst_call is None:
        _inst_call = _make_inst_kernel()
        _main_call = _make_main_kernel()
    hid_t = jnp.transpose(hidden_states, (1, 2, 0))
    sid_t = jnp.transpose(session_ids.astype(jnp.int32), (1, 0))
    itab_t = jnp.transpose(instrument_table, (1, 0))
    stab_t = jnp.transpose(session_table, (1, 0))
    iids = instrument_ids.astype(jnp.int32)
    iemb = _inst_call(itab_t, iids)
    out_t = _main_call(hid_t, sid_t, stab_t, iemb)
    return jnp.transpose(out_t, (2, 0, 1))
